# Initial kernel scaffold; baseline (speedup 1.0000x reference)
import functools

import jax
import jax.numpy as jnp
from jax import lax
from jax.experimental import pallas as pl
from jax.experimental.pallas import tpu as pltpu
from jax.experimental.pallas import tpu_sc as plsc

N_NODES = 10000
N_EDGES = 320000
D = 128

NC = 2
NS = 16
NW = NC * NS

EPW = N_EDGES // NW
CHUNK = 500
NITER = EPW // CHUNK

ROWS_PER_TILE = N_NODES // NS
ZBLK = 125

SELU_ALPHA = 1.6732632423543772
SELU_SCALE = 1.0507009873554805


def _matmul_body(x_ref, w_ref, o_ref):
    o_ref[...] = jnp.dot(x_ref[...], w_ref[...],
                         preferred_element_type=jnp.float32)


def _tc_matmul(features, W):
    grid = (N_NODES // 1000,)
    return pl.pallas_call(
        _matmul_body,
        grid=grid,
        in_specs=[
            pl.BlockSpec((1000, D), lambda i: (i, 0)),
            pl.BlockSpec((D, D), lambda i: (0, 0)),
        ],
        out_specs=pl.BlockSpec((1000, D), lambda i: (i, 0)),
        out_shape=jax.ShapeDtypeStruct((N_NODES, D), jnp.float32),
    )(features, W)


def _sc_body(out_hbm, src_hbm, dst_hbm, part_hbm,
             sidx_v, didx_v, rows_v, zero_v, agg_sh, sem):
    c = lax.axis_index("c")
    s = lax.axis_index("s")
    wid = s * NC + c

    def _zrow(i, _):
        for j in range(D // 16):
            zero_v[i, pl.ds(j * 16, 16)] = jnp.zeros((16,), jnp.float32)
        return 0
    lax.fori_loop(0, ZBLK, _zrow, 0)

    def _zcopy(k, _):
        pltpu.sync_copy(zero_v,
                        agg_sh.at[pl.ds(s * ROWS_PER_TILE + k * ZBLK, ZBLK)])
        return 0
    lax.fori_loop(0, ROWS_PER_TILE // ZBLK, _zcopy, 0)

    pltpu.sync_copy(src_hbm.at[wid], sidx_v)
    pltpu.sync_copy(dst_hbm.at[wid], didx_v)

    plsc.subcore_barrier()

    def _step(i, _):
        pltpu.async_copy(out_hbm.at[sidx_v.at[i]], rows_v, sem).wait()
        pltpu.sync_copy(rows_v, agg_sh.at[didx_v.at[i]], add=True)
        return 0
    lax.fori_loop(0, NITER, _step, 0)

    plsc.subcore_barrier()

    pltpu.sync_copy(agg_sh.at[pl.ds(s * ROWS_PER_TILE, ROWS_PER_TILE)],
                    part_hbm.at[c, pl.ds(s * ROWS_PER_TILE, ROWS_PER_TILE)])


def _sc_scatter(output, src3, dst3):
    mesh = plsc.VectorSubcoreMesh(core_axis_name="c", subcore_axis_name="s")
    f = pl.kernel(
        _sc_body,
        out_type=jax.ShapeDtypeStruct((NC, N_NODES, D), jnp.float32),
        mesh=mesh,
        scratch_types=[
            pltpu.VMEM((NITER, CHUNK), jnp.int32),
            pltpu.VMEM((NITER, CHUNK), jnp.int32),
            pltpu.VMEM((CHUNK, D), jnp.float32),
            pltpu.VMEM((ZBLK, D), jnp.float32),
            pltpu.VMEM_SHARED((N_NODES, D), jnp.float32),
            pltpu.SemaphoreType.DMA,
        ],
    )
    return f(output, src3, dst3)


def _epi_body(o_ref, p_ref, b_ref, sw_ref, out_ref):
    x = (o_ref[...] * sw_ref[...] + p_ref[0] + p_ref[1] + b_ref[...])
    out_ref[...] = jnp.where(
        x > 0, SELU_SCALE * x, SELU_SCALE * SELU_ALPHA * jnp.expm1(x))


def _tc_epilogue(output, partials, bias, skip_weight):
    grid = (N_NODES // 1000,)
    return pl.pallas_call(
        _epi_body,
        grid=grid,
        in_specs=[
            pl.BlockSpec((1000, D), lambda i: (i, 0)),
            pl.BlockSpec((NC, 1000, D), lambda i: (0, i, 0)),
            pl.BlockSpec((1, D), lambda i: (0, 0)),
            pl.BlockSpec((1, D), lambda i: (0, 0)),
        ],
        out_specs=pl.BlockSpec((1000, D), lambda i: (i, 0)),
        out_shape=jax.ShapeDtypeStruct((N_NODES, D), jnp.float32),
    )(features, partials, bias, skip_weight)


@jax.jit
def kernel(features, edge_index, W, bias, skip_weight):
    src3 = edge_index[0].reshape(NW, NITER, CHUNK)
    dst3 = edge_index[1].reshape(NW, NITER, CHUNK)
    output = _tc_matmul(features, W)
    partials = _sc_scatter(output, src3, dst3)
    return _tc_epilogue(output, partials,
                        bias.reshape(1, D), skip_weight.reshape(1, D))


# speedup vs baseline: 7.8772x; 7.8772x over previous
import functools

import jax
import jax.numpy as jnp
from jax import lax
from jax.experimental import pallas as pl
from jax.experimental.pallas import tpu as pltpu
from jax.experimental.pallas import tpu_sc as plsc

N_NODES = 10000
N_EDGES = 320000
D = 128

NC = 2
NS = 16
NW = NC * NS

EPW = N_EDGES // NW
CHUNK = 200
NITER = EPW // CHUNK

RBLK = 80
TILE_ROW_BASE = 640

SELU_ALPHA = 1.6732632423543772
SELU_SCALE = 1.0507009873554805


def _matmul_body(x_ref, w_ref, o_ref):
    o_ref[...] = jnp.dot(x_ref[...], w_ref[...],
                         preferred_element_type=jnp.float32)


def _tc_matmul(features, W):
    grid = (N_NODES // 1000,)
    return pl.pallas_call(
        _matmul_body,
        grid=grid,
        in_specs=[
            pl.BlockSpec((1000, D), lambda i: (i, 0)),
            pl.BlockSpec((D, D), lambda i: (0, 0)),
        ],
        out_specs=pl.BlockSpec((1000, D), lambda i: (i, 0)),
        out_shape=jax.ShapeDtypeStruct((N_NODES, D), jnp.float32),
    )(features, W)


def _sc_body(out_hbm, src_hbm, dst_hbm, part_hbm,
             sidx_v, didx_v, rows_v, agg_sh, sem):
    c = lax.axis_index("c")
    s = lax.axis_index("s")
    wid = s * NC + c

    row_base = s * TILE_ROW_BASE
    nblk = jnp.where(s < NS - 1, 8, 5)

    def _zrow(i, _):
        for j in range(D // 16):
            rows_v[i, pl.ds(j * 16, 16)] = jnp.zeros((16,), jnp.float32)
        return 0
    lax.fori_loop(0, RBLK, _zrow, 0)

    def _zcopy(k, _):
        pltpu.sync_copy(rows_v.at[pl.ds(0, RBLK)],
                        agg_sh.at[pl.ds(row_base + k * RBLK, RBLK)])
        return 0
    lax.fori_loop(0, nblk, _zcopy, 0)

    plsc.subcore_barrier()

    def _step(i, _):
        pltpu.sync_copy(src_hbm.at[wid, i], sidx_v)
        pltpu.sync_copy(dst_hbm.at[wid, i], didx_v)
        pltpu.async_copy(out_hbm.at[sidx_v], rows_v, sem).wait()
        pltpu.sync_copy(rows_v, agg_sh.at[didx_v], add=True)
        return 0
    lax.fori_loop(0, NITER, _step, 0)

    plsc.subcore_barrier()

    def _wcopy(k, _):
        off = row_base + k * RBLK
        pltpu.sync_copy(agg_sh.at[pl.ds(off, RBLK)],
                        part_hbm.at[c, pl.ds(off, RBLK)])
        return 0
    lax.fori_loop(0, nblk, _wcopy, 0)


def _sc_scatter(output, src3, dst3):
    mesh = plsc.VectorSubcoreMesh(core_axis_name="c", subcore_axis_name="s")
    f = pl.kernel(
        _sc_body,
        out_type=jax.ShapeDtypeStruct((NC, N_NODES, D), jnp.float32),
        mesh=mesh,
        compiler_params=pltpu.CompilerParams(use_tc_tiling_on_sc=False),
        scratch_types=[
            pltpu.VMEM((CHUNK,), jnp.int32),
            pltpu.VMEM((CHUNK,), jnp.int32),
            pltpu.VMEM((CHUNK, D), jnp.float32),
            pltpu.VMEM_SHARED((N_NODES, D), jnp.float32),
            pltpu.SemaphoreType.DMA,
        ],
    )
    return f(output, src3, dst3)


def _epi_body(o_ref, p_ref, b_ref, sw_ref, out_ref):
    x = (o_ref[...] * sw_ref[...] + p_ref[0] + p_ref[1] + b_ref[...])
    out_ref[...] = jnp.where(
        x > 0, SELU_SCALE * x, SELU_SCALE * SELU_ALPHA * (jnp.exp(x) - 1.0))


def _tc_epilogue(output, partials, bias, skip_weight):
    grid = (N_NODES // 1000,)
    return pl.pallas_call(
        _epi_body,
        grid=grid,
        in_specs=[
            pl.BlockSpec((1000, D), lambda i: (i, 0)),
            pl.BlockSpec((NC, 1000, D), lambda i: (0, i, 0)),
            pl.BlockSpec((1, D), lambda i: (0, 0)),
            pl.BlockSpec((1, D), lambda i: (0, 0)),
        ],
        out_specs=pl.BlockSpec((1000, D), lambda i: (i, 0)),
        out_shape=jax.ShapeDtypeStruct((N_NODES, D), jnp.float32),
    )(output, partials, bias, skip_weight)


@jax.jit
def kernel(features, edge_index, W, bias, skip_weight):
    src3 = edge_index[0].reshape(NW, NITER, CHUNK)
    dst3 = edge_index[1].reshape(NW, NITER, CHUNK)
    output = _tc_matmul(features, W)
    partials = _sc_scatter(output, src3, dst3)
    return _tc_epilogue(output, partials,
                        bias.reshape(1, D), skip_weight.reshape(1, D))


# speedup vs baseline: 9.9403x; 1.2619x over previous
import jax
import jax.numpy as jnp
from jax import lax
from jax.experimental import pallas as pl
from jax.experimental.pallas import tpu as pltpu
from jax.experimental.pallas import tpu_sc as plsc

N_NODES = 10000
N_EDGES = 320000
D = 128
DH = D // 2

NC = 2
NS = 16

EPT = N_EDGES // NS
CHUNK = 400
NITER = EPT // CHUNK

RBLK = 80
TILE_ROW_BASE = 640

SELU_ALPHA = 1.6732632423543772
SELU_SCALE = 1.0507009873554805


def _matmul_body(x_ref, w_ref, o_ref):
    res = jnp.dot(x_ref[...], w_ref[...], preferred_element_type=jnp.float32)
    o_ref[0] = res[:, :DH]
    o_ref[1] = res[:, DH:]


def _tc_matmul(features, W):
    grid = (N_NODES // 1000,)
    return pl.pallas_call(
        _matmul_body,
        grid=grid,
        in_specs=[
            pl.BlockSpec((1000, D), lambda i: (i, 0)),
            pl.BlockSpec((D, D), lambda i: (0, 0)),
        ],
        out_specs=pl.BlockSpec((NC, 1000, DH), lambda i: (0, i, 0)),
        out_shape=jax.ShapeDtypeStruct((NC, N_NODES, DH), jnp.float32),
    )(features, W)


def _sc_body(out_hbm, src_hbm, dst_hbm, agg_hbm,
             sidx0, sidx1, didx0, didx1, rows0, rows1, agg_sh, sem0, sem1):
    c = lax.axis_index("c")
    s = lax.axis_index("s")

    row_base = s * TILE_ROW_BASE
    nblk = jnp.where(s < NS - 1, 8, 5)

    def _zrow(i, _):
        for j in range(DH // 16):
            rows0[i, pl.ds(j * 16, 16)] = jnp.zeros((16,), jnp.float32)
        return 0
    lax.fori_loop(0, RBLK, _zrow, 0)

    def _zcopy(k, _):
        pltpu.sync_copy(rows0.at[pl.ds(0, RBLK)],
                        agg_sh.at[pl.ds(row_base + k * RBLK, RBLK)])
        return 0
    lax.fori_loop(0, nblk, _zcopy, 0)

    plsc.subcore_barrier()

    sidx = (sidx0, sidx1)
    didx = (didx0, didx1)
    rows = (rows0, rows1)
    sems = (sem0, sem1)

    def _start(i, b):
        pltpu.sync_copy(src_hbm.at[s, i], sidx[b])
        pltpu.sync_copy(dst_hbm.at[s, i], didx[b])
        pltpu.async_copy(out_hbm.at[c].at[sidx[b]], rows[b], sems[b])

    _start(0, 0)

    def _outer(g, _):
        for b in range(2):
            i = 2 * g + b

            @pl.when(i + 1 < NITER)
            def _():
                _start(i + 1, (b + 1) % 2)

            pltpu.make_async_copy(out_hbm.at[c].at[sidx[b]], rows[b],
                                  sems[b]).wait()
            pltpu.sync_copy(rows[b], agg_sh.at[didx[b]], add=True)
        return 0
    lax.fori_loop(0, NITER // 2, _outer, 0)

    plsc.subcore_barrier()

    def _wcopy(k, _):
        off = row_base + k * RBLK
        pltpu.sync_copy(agg_sh.at[pl.ds(off, RBLK)],
                        agg_hbm.at[c, pl.ds(off, RBLK)])
        return 0
    lax.fori_loop(0, nblk, _wcopy, 0)


def _sc_scatter(out2, src2, dst2):
    mesh = plsc.VectorSubcoreMesh(core_axis_name="c", subcore_axis_name="s")
    f = pl.kernel(
        _sc_body,
        out_type=jax.ShapeDtypeStruct((NC, N_NODES, DH), jnp.float32),
        mesh=mesh,
        compiler_params=pltpu.CompilerParams(use_tc_tiling_on_sc=False),
        scratch_types=[
            pltpu.VMEM((CHUNK,), jnp.int32),
            pltpu.VMEM((CHUNK,), jnp.int32),
            pltpu.VMEM((CHUNK,), jnp.int32),
            pltpu.VMEM((CHUNK,), jnp.int32),
            pltpu.VMEM((CHUNK, DH), jnp.float32),
            pltpu.VMEM((CHUNK, DH), jnp.float32),
            pltpu.VMEM_SHARED((N_NODES, DH), jnp.float32),
            pltpu.SemaphoreType.DMA,
            pltpu.SemaphoreType.DMA,
        ],
    )
    return f(out2, src2, dst2)


def _epi_body(o_ref, a_ref, b_ref, sw_ref, out_ref):
    o = jnp.concatenate([o_ref[0], o_ref[1]], axis=-1)
    a = jnp.concatenate([a_ref[0], a_ref[1]], axis=-1)
    x = o * sw_ref[...] + a + b_ref[...]
    out_ref[...] = jnp.where(
        x > 0, SELU_SCALE * x, SELU_SCALE * SELU_ALPHA * (jnp.exp(x) - 1.0))


def _tc_epilogue(out2, agg2, bias, skip_weight):
    grid = (N_NODES // 1000,)
    return pl.pallas_call(
        _epi_body,
        grid=grid,
        in_specs=[
            pl.BlockSpec((NC, 1000, DH), lambda i: (0, i, 0)),
            pl.BlockSpec((NC, 1000, DH), lambda i: (0, i, 0)),
            pl.BlockSpec((1, D), lambda i: (0, 0)),
            pl.BlockSpec((1, D), lambda i: (0, 0)),
        ],
        out_specs=pl.BlockSpec((1000, D), lambda i: (i, 0)),
        out_shape=jax.ShapeDtypeStruct((N_NODES, D), jnp.float32),
    )(out2, agg2, bias, skip_weight)


@jax.jit
def kernel(features, edge_index, W, bias, skip_weight):
    src2 = edge_index[0].reshape(NS, NITER, CHUNK)
    dst2 = edge_index[1].reshape(NS, NITER, CHUNK)
    out2 = _tc_matmul(features, W)
    agg2 = _sc_scatter(out2, src2, dst2)
    return _tc_epilogue(out2, agg2,
                        bias.reshape(1, D), skip_weight.reshape(1, D))


# speedup vs baseline: 10.9207x; 1.0986x over previous
import jax
import jax.numpy as jnp
from jax import lax
from jax.experimental import pallas as pl
from jax.experimental.pallas import tpu as pltpu
from jax.experimental.pallas import tpu_sc as plsc

N_NODES = 10000
N_EDGES = 320000
D = 128
DH = D // 2

NC = 2
NS = 16
NBUF = 3

EPT = N_EDGES // NS
CHUNK = 400
NITER = EPT // CHUNK

RBLK = 80
TILE_ROW_BASE = 640

SELU_ALPHA = 1.6732632423543772
SELU_SCALE = 1.0507009873554805


def _sc_body(feat_hbm, eidx_hbm, agg_hbm,
             idx0, idx1, idx2, rows0, rows1, rows2, agg_sh,
             sem0, sem1, sem2):
    c = lax.axis_index("c")
    s = lax.axis_index("s")

    row_base = s * TILE_ROW_BASE
    nblk = jnp.where(s < NS - 1, 8, 5)

    def _zrow(i, _):
        for j in range(DH // 16):
            rows0[i, pl.ds(j * 16, 16)] = jnp.zeros((16,), jnp.float32)
        return 0
    lax.fori_loop(0, RBLK, _zrow, 0)

    def _zcopy(k, _):
        pltpu.sync_copy(rows0.at[pl.ds(0, RBLK)],
                        agg_sh.at[pl.ds(row_base + k * RBLK, RBLK)])
        return 0
    lax.fori_loop(0, nblk, _zcopy, 0)

    plsc.subcore_barrier()

    idx = (idx0, idx1, idx2)
    rows = (rows0, rows1, rows2)
    sems = (sem0, sem1, sem2)

    def _start(i, b):
        pltpu.sync_copy(eidx_hbm.at[s, i], idx[b])
        pltpu.async_copy(feat_hbm.at[c].at[idx[b].at[0]], rows[b], sems[b])

    _start(0, 0)
    _start(1, 1)

    def _outer(g, _):
        for b in range(NBUF):
            i = NBUF * g + b

            @pl.when(i + 2 < NITER)
            def _():
                _start(i + 2, (b + 2) % NBUF)

            @pl.when(i < NITER)
            def _():
                pltpu.make_async_copy(feat_hbm.at[c].at[idx[b].at[0]],
                                      rows[b], sems[b]).wait()
                pltpu.sync_copy(rows[b], agg_sh.at[idx[b].at[1]], add=True)
        return 0
    lax.fori_loop(0, (NITER + NBUF - 1) // NBUF, _outer, 0)

    plsc.subcore_barrier()

    def _wcopy(k, _):
        off = row_base + k * RBLK
        pltpu.sync_copy(agg_sh.at[pl.ds(off, RBLK)],
                        agg_hbm.at[c, pl.ds(off, RBLK)])
        return 0
    lax.fori_loop(0, nblk, _wcopy, 0)


def _sc_scatter(feat2, eidx):
    mesh = plsc.VectorSubcoreMesh(core_axis_name="c", subcore_axis_name="s")
    f = pl.kernel(
        _sc_body,
        out_type=jax.ShapeDtypeStruct((NC, N_NODES, DH), jnp.float32),
        mesh=mesh,
        compiler_params=pltpu.CompilerParams(use_tc_tiling_on_sc=False),
        scratch_types=[
            pltpu.VMEM((2, CHUNK), jnp.int32),
            pltpu.VMEM((2, CHUNK), jnp.int32),
            pltpu.VMEM((2, CHUNK), jnp.int32),
            pltpu.VMEM((CHUNK, DH), jnp.float32),
            pltpu.VMEM((CHUNK, DH), jnp.float32),
            pltpu.VMEM((CHUNK, DH), jnp.float32),
            pltpu.VMEM_SHARED((N_NODES, DH), jnp.float32),
            pltpu.SemaphoreType.DMA,
            pltpu.SemaphoreType.DMA,
            pltpu.SemaphoreType.DMA,
        ],
    )
    return f(feat2, eidx)


def _final_body(f_ref, a_ref, wsw_ref, w_ref, b_ref, out_ref):
    a = jnp.concatenate([a_ref[0], a_ref[1]], axis=-1)
    x = (jnp.dot(f_ref[...], wsw_ref[...], preferred_element_type=jnp.float32)
         + jnp.dot(a, w_ref[...], preferred_element_type=jnp.float32)
         + b_ref[...])
    out_ref[...] = jnp.where(
        x > 0, SELU_SCALE * x, SELU_SCALE * SELU_ALPHA * (jnp.exp(x) - 1.0))


def _tc_final(features, agg2, Wsw, W, bias):
    grid = (N_NODES // 1000,)
    return pl.pallas_call(
        _final_body,
        grid=grid,
        in_specs=[
            pl.BlockSpec((1000, D), lambda i: (i, 0)),
            pl.BlockSpec((NC, 1000, DH), lambda i: (0, i, 0)),
            pl.BlockSpec((D, D), lambda i: (0, 0)),
            pl.BlockSpec((D, D), lambda i: (0, 0)),
            pl.BlockSpec((1, D), lambda i: (0, 0)),
        ],
        out_specs=pl.BlockSpec((1000, D), lambda i: (i, 0)),
        out_shape=jax.ShapeDtypeStruct((N_NODES, D), jnp.float32),
    )(features, agg2, Wsw, W, bias)


@jax.jit
def kernel(features, edge_index, W, bias, skip_weight):
    feat2 = features.reshape(N_NODES, NC, DH).swapaxes(0, 1)
    eidx = edge_index.reshape(2, NS, NITER, CHUNK).transpose(1, 2, 0, 3)
    agg2 = _sc_scatter(feat2, eidx)
    Wsw = W * skip_weight[None, :]
    return _tc_final(features, agg2, Wsw, W, bias.reshape(1, D))


# speedup vs baseline: 12.1159x; 1.1094x over previous
import jax
import jax.numpy as jnp
from jax import lax
from jax.experimental import pallas as pl
from jax.experimental.pallas import tpu as pltpu
from jax.experimental.pallas import tpu_sc as plsc

N_NODES = 10000
N_EDGES = 320000
D = 128
DH = D // 2

NC = 2
NS = 16
NBUF = 3

EPT = N_EDGES // NS
CHUNK = 400
NITER = EPT // CHUNK

RBLK = 80
TILE_ROW_BASE = 640

SELU_ALPHA = 1.6732632423543772
SELU_SCALE = 1.0507009873554805


def _sc_body(feat_hbm, eidx_hbm, agg_hbm,
             idx0, idx1, idx2, rows0, rows1, rows2, agg_sh,
             sem0, sem1, sem2):
    c = lax.axis_index("c")
    s = lax.axis_index("s")

    row_base = s * TILE_ROW_BASE
    nblk = jnp.where(s < NS - 1, 8, 5)

    def _zrow(i, _):
        for j in range(DH // 16):
            rows0[i, pl.ds(j * 16, 16)] = jnp.zeros((16,), jnp.float32)
        return 0
    lax.fori_loop(0, RBLK, _zrow, 0)

    def _zcopy(k, _):
        pltpu.sync_copy(rows0.at[pl.ds(0, RBLK)],
                        agg_sh.at[pl.ds(row_base + k * RBLK, RBLK)])
        return 0
    lax.fori_loop(0, nblk, _zcopy, 0)

    plsc.subcore_barrier()

    idx = (idx0, idx1, idx2)
    rows = (rows0, rows1, rows2)
    sems = (sem0, sem1, sem2)

    col = c * DH

    def _start(i, b):
        pltpu.sync_copy(eidx_hbm.at[0, s, i], idx[b].at[0])
        pltpu.sync_copy(eidx_hbm.at[1, s, i], idx[b].at[1])

        def _fix(j, _):
            v = idx[b][0, pl.ds(j * 16, 16)]
            idx[b][0, pl.ds(j * 16, 16)] = v + v + c
            return 0
        lax.fori_loop(0, CHUNK // 16, _fix, 0)
        pltpu.async_copy(feat_hbm.at[idx[b].at[0]], rows[b], sems[b])

    _start(0, 0)
    _start(1, 1)

    def _outer(g, _):
        for b in range(NBUF):
            i = NBUF * g + b

            @pl.when(i + 2 < NITER)
            def _():
                _start(i + 2, (b + 2) % NBUF)

            @pl.when(i < NITER)
            def _():
                pltpu.make_async_copy(feat_hbm.at[idx[b].at[0]],
                                      rows[b], sems[b]).wait()
                pltpu.sync_copy(rows[b], agg_sh.at[idx[b].at[1]], add=True)
        return 0
    lax.fori_loop(0, (NITER + NBUF - 1) // NBUF, _outer, 0)

    plsc.subcore_barrier()

    def _wcopy(k, _):
        off = row_base + k * RBLK
        pltpu.sync_copy(agg_sh.at[pl.ds(off, RBLK)],
                        agg_hbm.at[pl.ds(off, RBLK), pl.ds(col, DH)])
        return 0
    lax.fori_loop(0, nblk, _wcopy, 0)


def _sc_scatter(features, eidx):
    mesh = plsc.VectorSubcoreMesh(core_axis_name="c", subcore_axis_name="s")
    f = pl.kernel(
        _sc_body,
        out_type=jax.ShapeDtypeStruct((N_NODES, D), jnp.float32),
        mesh=mesh,
        compiler_params=pltpu.CompilerParams(use_tc_tiling_on_sc=False),
        scratch_types=[
            pltpu.VMEM((2, CHUNK), jnp.int32),
            pltpu.VMEM((2, CHUNK), jnp.int32),
            pltpu.VMEM((2, CHUNK), jnp.int32),
            pltpu.VMEM((CHUNK, DH), jnp.float32),
            pltpu.VMEM((CHUNK, DH), jnp.float32),
            pltpu.VMEM((CHUNK, DH), jnp.float32),
            pltpu.VMEM_SHARED((N_NODES, DH), jnp.float32),
            pltpu.SemaphoreType.DMA,
            pltpu.SemaphoreType.DMA,
            pltpu.SemaphoreType.DMA,
        ],
    )
    return f(features, eidx)


def _final_body(f_ref, a_ref, w_ref, sw_ref, b_ref, out_ref):
    wsw = w_ref[...] * sw_ref[...]
    x = (jnp.dot(f_ref[...], wsw, preferred_element_type=jnp.float32)
         + jnp.dot(a_ref[...], w_ref[...], preferred_element_type=jnp.float32)
         + b_ref[...])
    out_ref[...] = jnp.where(
        x > 0, SELU_SCALE * x, SELU_SCALE * SELU_ALPHA * (jnp.exp(x) - 1.0))


def _tc_final(features, agg, W, skip_weight, bias):
    grid = (N_NODES // 1000,)
    return pl.pallas_call(
        _final_body,
        grid=grid,
        in_specs=[
            pl.BlockSpec((1000, D), lambda i: (i, 0)),
            pl.BlockSpec((1000, D), lambda i: (i, 0)),
            pl.BlockSpec((D, D), lambda i: (0, 0)),
            pl.BlockSpec((1, D), lambda i: (0, 0)),
            pl.BlockSpec((1, D), lambda i: (0, 0)),
        ],
        out_specs=pl.BlockSpec((1000, D), lambda i: (i, 0)),
        out_shape=jax.ShapeDtypeStruct((N_NODES, D), jnp.float32),
    )(features, agg, W, skip_weight, bias)


@jax.jit
def kernel(features, edge_index, W, bias, skip_weight):
    eidx = edge_index.reshape(2, NS, NITER, CHUNK)
    feat_r = features.reshape(N_NODES * NC, DH)
    agg = _sc_scatter(feat_r, eidx)
    return _tc_final(features, agg, W,
                     skip_weight.reshape(1, D), bias.reshape(1, D))


# speedup vs baseline: 12.4212x; 1.0252x over previous
import jax
import jax.numpy as jnp
from jax import lax
from jax.experimental import pallas as pl
from jax.experimental.pallas import tpu as pltpu
from jax.experimental.pallas import tpu_sc as plsc

N_NODES = 10000
N_EDGES = 320000
D = 128
DH = D // 2

NC = 2
NS = 16
NBUF = 3

EPT = N_EDGES // NS
CHUNK = 400
NITER = EPT // CHUNK

RBLK = 80
TILE_ROW_BASE = 640

SELU_ALPHA = 1.6732632423543772
SELU_SCALE = 1.0507009873554805


def _sc_body(feat_hbm, eidx_hbm, agg_hbm,
             idx0, idx1, idx2, rows0, rows1, rows2, agg_sh,
             sem0, sem1, sem2):
    c = lax.axis_index("c")
    s = lax.axis_index("s")

    row_base = s * TILE_ROW_BASE
    nblk = jnp.where(s < NS - 1, 8, 5)

    def _zrow(i, _):
        for j in range(DH // 16):
            rows0[i, pl.ds(j * 16, 16)] = jnp.zeros((16,), jnp.float32)
        return 0
    lax.fori_loop(0, RBLK, _zrow, 0)

    def _zcopy(k, _):
        pltpu.sync_copy(rows0.at[pl.ds(0, RBLK)],
                        agg_sh.at[pl.ds(row_base + k * RBLK, RBLK)])
        return 0
    lax.fori_loop(0, nblk, _zcopy, 0)

    plsc.subcore_barrier()

    idx = (idx0, idx1, idx2)
    rows = (rows0, rows1, rows2)
    sems = (sem0, sem1, sem2)

    col = c * DH
    feat_c = feat_hbm.at[pl.ds(c, NC * N_NODES - 1)]

    def _start(i, b):
        pltpu.sync_copy(eidx_hbm.at[0, s, i], idx[b].at[0])
        pltpu.sync_copy(eidx_hbm.at[1, s, i], idx[b].at[1])
        pltpu.async_copy(feat_c.at[idx[b].at[0]], rows[b], sems[b])

    _start(0, 0)
    _start(1, 1)

    def _outer(g, _):
        for b in range(NBUF):
            i = NBUF * g + b

            @pl.when(i + 2 < NITER)
            def _():
                _start(i + 2, (b + 2) % NBUF)

            @pl.when(i < NITER)
            def _():
                pltpu.make_async_copy(feat_c.at[idx[b].at[0]],
                                      rows[b], sems[b]).wait()
                pltpu.sync_copy(rows[b], agg_sh.at[idx[b].at[1]], add=True)
        return 0
    lax.fori_loop(0, (NITER + NBUF - 1) // NBUF, _outer, 0)

    plsc.subcore_barrier()

    def _wcopy(k, _):
        off = row_base + k * RBLK
        pltpu.sync_copy(agg_sh.at[pl.ds(off, RBLK)],
                        agg_hbm.at[pl.ds(off, RBLK), pl.ds(col, DH)])
        return 0
    lax.fori_loop(0, nblk, _wcopy, 0)


def _sc_scatter(features, eidx):
    mesh = plsc.VectorSubcoreMesh(core_axis_name="c", subcore_axis_name="s")
    f = pl.kernel(
        _sc_body,
        out_type=jax.ShapeDtypeStruct((N_NODES, D), jnp.float32),
        mesh=mesh,
        compiler_params=pltpu.CompilerParams(use_tc_tiling_on_sc=False),
        scratch_types=[
            pltpu.VMEM((2, CHUNK), jnp.int32),
            pltpu.VMEM((2, CHUNK), jnp.int32),
            pltpu.VMEM((2, CHUNK), jnp.int32),
            pltpu.VMEM((CHUNK, DH), jnp.float32),
            pltpu.VMEM((CHUNK, DH), jnp.float32),
            pltpu.VMEM((CHUNK, DH), jnp.float32),
            pltpu.VMEM_SHARED((N_NODES, DH), jnp.float32),
            pltpu.SemaphoreType.DMA,
            pltpu.SemaphoreType.DMA,
            pltpu.SemaphoreType.DMA,
        ],
    )
    return f(features, eidx)


def _final_body(f_ref, a_ref, w_ref, sw_ref, b_ref, out_ref):
    wsw = w_ref[...] * sw_ref[...]
    x = (jnp.dot(f_ref[...], wsw, preferred_element_type=jnp.float32)
         + jnp.dot(a_ref[...], w_ref[...], preferred_element_type=jnp.float32)
         + b_ref[...])
    out_ref[...] = jnp.where(
        x > 0, SELU_SCALE * x, SELU_SCALE * SELU_ALPHA * (jnp.exp(x) - 1.0))


def _tc_final(features, agg, W, skip_weight, bias):
    grid = (N_NODES // 1000,)
    return pl.pallas_call(
        _final_body,
        grid=grid,
        in_specs=[
            pl.BlockSpec((1000, D), lambda i: (i, 0)),
            pl.BlockSpec((1000, D), lambda i: (i, 0)),
            pl.BlockSpec((D, D), lambda i: (0, 0)),
            pl.BlockSpec((1, D), lambda i: (0, 0)),
            pl.BlockSpec((1, D), lambda i: (0, 0)),
        ],
        out_specs=pl.BlockSpec((1000, D), lambda i: (i, 0)),
        out_shape=jax.ShapeDtypeStruct((N_NODES, D), jnp.float32),
    )(features, agg, W, skip_weight, bias)


@jax.jit
def kernel(features, edge_index, W, bias, skip_weight):
    eidx = (edge_index * jnp.array([[2], [1]], jnp.int32)
            ).reshape(2, NS, NITER, CHUNK)
    feat_r = features.reshape(N_NODES * NC, DH)
    agg = _sc_scatter(feat_r, eidx)
    return _tc_final(features, agg, W,
                     skip_weight.reshape(1, D), bias.reshape(1, D))


# speedup vs baseline: 13.0416x; 1.0499x over previous
import jax
import jax.numpy as jnp
from jax import lax
from jax.experimental import pallas as pl
from jax.experimental.pallas import tpu as pltpu
from jax.experimental.pallas import tpu_sc as plsc

N_NODES = 10000
N_EDGES = 320000
D = 128
DH = D // 2

NC = 2
NS = 16
NBUF = 3

EPT = N_EDGES // NS
CHUNK = 400
NITER = EPT // CHUNK

RBLK = 80
TILE_ROW_BASE = 640

SELU_ALPHA = 1.6732632423543772
SELU_SCALE = 1.0507009873554805


def _sc_body(feat_hbm, eidx_hbm, agg_hbm,
             idx0, idx1, idx2, rows0, rows1, rows2, agg_sh,
             sem0, sem1, sem2, isem0, isem1, isem2):
    c = lax.axis_index("c")
    s = lax.axis_index("s")

    row_base = s * TILE_ROW_BASE
    nblk = jnp.where(s < NS - 1, 8, 5)

    def _zrow(i, _):
        for j in range(DH // 16):
            rows0[i, pl.ds(j * 16, 16)] = jnp.zeros((16,), jnp.float32)
        return 0
    lax.fori_loop(0, RBLK, _zrow, 0)

    def _zcopy(k, _):
        pltpu.sync_copy(rows0.at[pl.ds(0, RBLK)],
                        agg_sh.at[pl.ds(row_base + k * RBLK, RBLK)])
        return 0
    lax.fori_loop(0, nblk, _zcopy, 0)

    plsc.subcore_barrier()

    idx = (idx0, idx1, idx2)
    rows = (rows0, rows1, rows2)
    sems = (sem0, sem1, sem2)
    isems = (isem0, isem1, isem2)

    col = c * DH
    feat_c = feat_hbm.at[pl.ds(c, NC * N_NODES - 1)]

    def _fire_idx(i, b):
        pltpu.async_copy(eidx_hbm.at[s, i], idx[b], isems[b])

    def _fire_gather(i, b):
        pltpu.make_async_copy(eidx_hbm.at[s, i], idx[b], isems[b]).wait()
        pltpu.async_copy(feat_c.at[idx[b].at[0]], rows[b], sems[b])

    _fire_idx(0, 0)
    _fire_idx(1, 1)
    _fire_gather(0, 0)

    def _outer(g, _):
        for b in range(NBUF):
            i = NBUF * g + b

            @pl.when(i + 2 < NITER)
            def _():
                _fire_idx(i + 2, (b + 2) % NBUF)

            @pl.when(i + 1 < NITER)
            def _():
                _fire_gather(i + 1, (b + 1) % NBUF)

            @pl.when(i < NITER)
            def _():
                pltpu.make_async_copy(feat_c.at[idx[b].at[0]],
                                      rows[b], sems[b]).wait()
                pltpu.sync_copy(rows[b], agg_sh.at[idx[b].at[1]], add=True)
        return 0
    lax.fori_loop(0, (NITER + NBUF - 1) // NBUF, _outer, 0)

    plsc.subcore_barrier()

    def _wcopy(k, _):
        off = row_base + k * RBLK
        pltpu.sync_copy(agg_sh.at[pl.ds(off, RBLK)],
                        agg_hbm.at[pl.ds(off, RBLK), pl.ds(col, DH)])
        return 0
    lax.fori_loop(0, nblk, _wcopy, 0)


def _sc_scatter(features, eidx):
    mesh = plsc.VectorSubcoreMesh(core_axis_name="c", subcore_axis_name="s")
    f = pl.kernel(
        _sc_body,
        out_type=jax.ShapeDtypeStruct((N_NODES, D), jnp.float32),
        mesh=mesh,
        compiler_params=pltpu.CompilerParams(use_tc_tiling_on_sc=False),
        scratch_types=[
            pltpu.VMEM((2, CHUNK), jnp.int32),
            pltpu.VMEM((2, CHUNK), jnp.int32),
            pltpu.VMEM((2, CHUNK), jnp.int32),
            pltpu.VMEM((CHUNK, DH), jnp.float32),
            pltpu.VMEM((CHUNK, DH), jnp.float32),
            pltpu.VMEM((CHUNK, DH), jnp.float32),
            pltpu.VMEM_SHARED((N_NODES, DH), jnp.float32),
            pltpu.SemaphoreType.DMA,
            pltpu.SemaphoreType.DMA,
            pltpu.SemaphoreType.DMA,
            pltpu.SemaphoreType.DMA,
            pltpu.SemaphoreType.DMA,
            pltpu.SemaphoreType.DMA,
        ],
    )
    return f(features, eidx)


def _final_body(f_ref, a_ref, w_ref, sw_ref, b_ref, out_ref):
    wsw = w_ref[...] * sw_ref[...]
    x = (jnp.dot(f_ref[...], wsw, preferred_element_type=jnp.float32)
         + jnp.dot(a_ref[...], w_ref[...], preferred_element_type=jnp.float32)
         + b_ref[...])
    out_ref[...] = jnp.where(
        x > 0, SELU_SCALE * x, SELU_SCALE * SELU_ALPHA * (jnp.exp(x) - 1.0))


def _tc_final(features, agg, W, skip_weight, bias):
    grid = (N_NODES // 1000,)
    return pl.pallas_call(
        _final_body,
        grid=grid,
        in_specs=[
            pl.BlockSpec((1000, D), lambda i: (i, 0)),
            pl.BlockSpec((1000, D), lambda i: (i, 0)),
            pl.BlockSpec((D, D), lambda i: (0, 0)),
            pl.BlockSpec((1, D), lambda i: (0, 0)),
            pl.BlockSpec((1, D), lambda i: (0, 0)),
        ],
        out_specs=pl.BlockSpec((1000, D), lambda i: (i, 0)),
        out_shape=jax.ShapeDtypeStruct((N_NODES, D), jnp.float32),
    )(features, agg, W, skip_weight, bias)


@jax.jit
def kernel(features, edge_index, W, bias, skip_weight):
    eidx = (edge_index * jnp.array([[2], [1]], jnp.int32)
            ).reshape(2, NS, NITER, CHUNK).transpose(1, 2, 0, 3)
    feat_r = features.reshape(N_NODES * NC, DH)
    agg = _sc_scatter(feat_r, eidx)
    return _tc_final(features, agg, W,
                     skip_weight.reshape(1, D), bias.reshape(1, D))


# speedup vs baseline: 13.0987x; 1.0044x over previous
import jax
import jax.numpy as jnp
from jax import lax
from jax.experimental import pallas as pl
from jax.experimental.pallas import tpu as pltpu
from jax.experimental.pallas import tpu_sc as plsc

N_NODES = 10000
N_EDGES = 320000
D = 128
DH = D // 2

NC = 2
NS = 16
NBUF = 3

EPT = N_EDGES // NS
CHUNK = 400
NITER = EPT // CHUNK

RBLK = 80
TILE_ROW_BASE = 640

SELU_ALPHA = 1.6732632423543772
SELU_SCALE = 1.0507009873554805


def _sc_body(feat_hbm, eidx_hbm, agg_hbm,
             idx0, idx1, idx2, rows0, rows1, rows2, agg_sh,
             sem0, sem1, sem2, isem0, isem1, isem2, wsem):
    c = lax.axis_index("c")
    s = lax.axis_index("s")

    row_base = s * TILE_ROW_BASE
    nblk = jnp.where(s < NS - 1, 8, 5)

    def _zrow(i, _):
        for j in range(DH // 16):
            rows0[i, pl.ds(j * 16, 16)] = jnp.zeros((16,), jnp.float32)
        return 0
    lax.fori_loop(0, RBLK, _zrow, 0)

    def _zcopy(k, _):
        pltpu.sync_copy(rows0.at[pl.ds(0, RBLK)],
                        agg_sh.at[pl.ds(row_base + k * RBLK, RBLK)])
        return 0
    lax.fori_loop(0, nblk, _zcopy, 0)

    plsc.subcore_barrier()

    idx = (idx0, idx1, idx2)
    rows = (rows0, rows1, rows2)
    sems = (sem0, sem1, sem2)
    isems = (isem0, isem1, isem2)

    col = c * DH
    feat_c = feat_hbm.at[pl.ds(c, NC * N_NODES - 1)]

    def _fire_idx(i, b):
        pltpu.async_copy(eidx_hbm.at[s, i], idx[b], isems[b])

    def _fire_gather(i, b):
        pltpu.make_async_copy(eidx_hbm.at[s, i], idx[b], isems[b]).wait()
        pltpu.async_copy(feat_c.at[idx[b].at[0]], rows[b], sems[b])

    _fire_idx(0, 0)
    _fire_idx(1, 1)
    _fire_gather(0, 0)

    def _outer(g, _):
        for b in range(NBUF):
            i = NBUF * g + b

            @pl.when(i + 1 < NITER)
            def _():
                _fire_gather(i + 1, (b + 1) % NBUF)

            @pl.when(i < NITER)
            def _():
                pltpu.make_async_copy(feat_c.at[idx[b].at[0]],
                                      rows[b], sems[b]).wait()
                pltpu.async_copy(rows[b], agg_sh.at[idx[b].at[1]], wsem,
                                 add=True)

            @pl.when((i >= 1) & (i < NITER + 1))
            def _():
                bp = (b + NBUF - 1) % NBUF
                pltpu.make_async_copy(rows[bp], agg_sh.at[idx[bp].at[1]],
                                      wsem).wait()

            @pl.when(i + 2 < NITER)
            def _():
                _fire_idx(i + 2, (b + 2) % NBUF)
        return 0
    lax.fori_loop(0, (NITER + NBUF + 1) // NBUF, _outer, 0)

    plsc.subcore_barrier()

    def _wcopy(k, _):
        off = row_base + k * RBLK
        pltpu.sync_copy(agg_sh.at[pl.ds(off, RBLK)],
                        agg_hbm.at[pl.ds(off, RBLK), pl.ds(col, DH)])
        return 0
    lax.fori_loop(0, nblk, _wcopy, 0)


def _sc_scatter(features, eidx):
    mesh = plsc.VectorSubcoreMesh(core_axis_name="c", subcore_axis_name="s")
    f = pl.kernel(
        _sc_body,
        out_type=jax.ShapeDtypeStruct((N_NODES, D), jnp.float32),
        mesh=mesh,
        compiler_params=pltpu.CompilerParams(use_tc_tiling_on_sc=False),
        scratch_types=[
            pltpu.VMEM((2, CHUNK), jnp.int32),
            pltpu.VMEM((2, CHUNK), jnp.int32),
            pltpu.VMEM((2, CHUNK), jnp.int32),
            pltpu.VMEM((CHUNK, DH), jnp.float32),
            pltpu.VMEM((CHUNK, DH), jnp.float32),
            pltpu.VMEM((CHUNK, DH), jnp.float32),
            pltpu.VMEM_SHARED((N_NODES, DH), jnp.float32),
            pltpu.SemaphoreType.DMA,
            pltpu.SemaphoreType.DMA,
            pltpu.SemaphoreType.DMA,
            pltpu.SemaphoreType.DMA,
            pltpu.SemaphoreType.DMA,
            pltpu.SemaphoreType.DMA,
            pltpu.SemaphoreType.DMA,
        ],
    )
    return f(features, eidx)


def _final_body(f_ref, a_ref, w_ref, sw_ref, b_ref, out_ref):
    wsw = w_ref[...] * sw_ref[...]
    x = (jnp.dot(f_ref[...], wsw, preferred_element_type=jnp.float32)
         + jnp.dot(a_ref[...], w_ref[...], preferred_element_type=jnp.float32)
         + b_ref[...])
    out_ref[...] = jnp.where(
        x > 0, SELU_SCALE * x, SELU_SCALE * SELU_ALPHA * (jnp.exp(x) - 1.0))


def _tc_final(features, agg, W, skip_weight, bias):
    grid = (N_NODES // 1000,)
    return pl.pallas_call(
        _final_body,
        grid=grid,
        in_specs=[
            pl.BlockSpec((1000, D), lambda i: (i, 0)),
            pl.BlockSpec((1000, D), lambda i: (i, 0)),
            pl.BlockSpec((D, D), lambda i: (0, 0)),
            pl.BlockSpec((1, D), lambda i: (0, 0)),
            pl.BlockSpec((1, D), lambda i: (0, 0)),
        ],
        out_specs=pl.BlockSpec((1000, D), lambda i: (i, 0)),
        out_shape=jax.ShapeDtypeStruct((N_NODES, D), jnp.float32),
    )(features, agg, W, skip_weight, bias)


@jax.jit
def kernel(features, edge_index, W, bias, skip_weight):
    eidx = (edge_index * jnp.array([[2], [1]], jnp.int32)
            ).reshape(2, NS, NITER, CHUNK).transpose(1, 2, 0, 3)
    feat_r = features.reshape(N_NODES * NC, DH)
    agg = _sc_scatter(feat_r, eidx)
    return _tc_final(features, agg, W,
                     skip_weight.reshape(1, D), bias.reshape(1, D))
